# speedup vs baseline: 1.0298x; 1.0298x over previous
import jax
import jax.numpy as jnp
from jax import lax
from jax.experimental import pallas as pl
from jax.experimental.pallas import tpu as pltpu
from jax.experimental.pallas import tpu_sc as plsc

B, S, D = 4, 8192, 128
FLAT = B * S
NC, NS = 2, 16
NW = NC * NS
PER_W = FLAT // NW
CHUNK = 128
N_CHUNKS = PER_W // CHUNK


def _emb_body(ids_hbm, tok_hbm, pos_hbm, out_hbm, idx_v, row_v, sem):
    wid = lax.axis_index("s") * NC + lax.axis_index("c")
    base_w = wid * PER_W

    def step(c, carry):
        base = base_w + c * CHUNK
        s0 = lax.rem(base, S)
        pltpu.sync_copy(ids_hbm.at[pl.ds(base, CHUNK)], idx_v)
        pltpu.sync_copy(pos_hbm.at[pl.ds(s0, CHUNK)], row_v)
        pltpu.async_copy(tok_hbm.at[idx_v], row_v, sem, add=True).wait()
        pltpu.sync_copy(row_v, out_hbm.at[pl.ds(base, CHUNK)])
        return carry

    lax.fori_loop(0, N_CHUNKS, step, 0)


def kernel(token_ids, tok_emb, pos_emb):
    ids = token_ids.reshape(FLAT).astype(jnp.int32)
    mesh = plsc.VectorSubcoreMesh(
        core_axis_name="c", subcore_axis_name="s",
        num_cores=NC, num_subcores=NS,
    )
    out = pl.kernel(
        _emb_body,
        out_type=jax.ShapeDtypeStruct((FLAT, D), jnp.float32),
        mesh=mesh,
        scratch_types=[
            pltpu.VMEM((CHUNK,), jnp.int32),
            pltpu.VMEM((CHUNK, D), jnp.float32),
            pltpu.SemaphoreType.DMA,
        ],
    )(ids, tok_emb, pos_emb)
    return out.reshape(B, S, D)


# speedup vs baseline: 1.2365x; 1.2007x over previous
import jax
import jax.numpy as jnp
from jax import lax
from jax.experimental import pallas as pl
from jax.experimental.pallas import tpu as pltpu
from jax.experimental.pallas import tpu_sc as plsc

B, S, D = 4, 8192, 128
FLAT = B * S
NC, NS = 2, 16
NW = NC * NS
PER_W = FLAT // NW
CHUNK = 128
N_CHUNKS = PER_W // CHUNK


NBUF = 2


def _emb_body(ids_hbm, tok_hbm, pos_hbm, out_hbm, idx_v, row_v,
              lsem, gsem, ssem):
    wid = lax.axis_index("s") * NC + lax.axis_index("c")
    base_w = wid * PER_W

    def start_loads(c):
        slot = c % NBUF
        base = base_w + c * CHUNK
        s0 = lax.rem(base, S)
        a = pltpu.async_copy(ids_hbm.at[pl.ds(base, CHUNK)], idx_v.at[slot], lsem)
        b = pltpu.async_copy(pos_hbm.at[pl.ds(s0, CHUNK)], row_v.at[slot], lsem)
        return a, b

    loads = {0: start_loads(0)}
    stores = {}
    for c in range(N_CHUNKS):
        slot = c % NBUF
        for d in loads.pop(c):
            d.wait()
        if c + 1 < N_CHUNKS:
            if c - 1 >= 0:
                stores.pop(c - 1).wait()
            loads[c + 1] = start_loads(c + 1)
        pltpu.async_copy(tok_hbm.at[idx_v.at[slot]], row_v.at[slot], gsem,
                         add=True).wait()
        base = base_w + c * CHUNK
        stores[c] = pltpu.async_copy(row_v.at[slot], out_hbm.at[pl.ds(base, CHUNK)], ssem)
    for c in sorted(stores):
        stores.pop(c).wait()


def kernel(token_ids, tok_emb, pos_emb):
    ids = token_ids.reshape(FLAT).astype(jnp.int32)
    mesh = plsc.VectorSubcoreMesh(
        core_axis_name="c", subcore_axis_name="s",
        num_cores=NC, num_subcores=NS,
    )
    out = pl.kernel(
        _emb_body,
        out_type=jax.ShapeDtypeStruct((FLAT, D), jnp.float32),
        mesh=mesh,
        scratch_types=[
            pltpu.VMEM((NBUF, CHUNK), jnp.int32),
            pltpu.VMEM((NBUF, CHUNK, D), jnp.float32),
            pltpu.SemaphoreType.DMA,
            pltpu.SemaphoreType.DMA,
            pltpu.SemaphoreType.DMA,
        ],
    )(ids, tok_emb, pos_emb)
    return out.reshape(B, S, D)


# speedup vs baseline: 1.3253x; 1.0718x over previous
import jax
import jax.numpy as jnp
from jax import lax
from jax.experimental import pallas as pl
from jax.experimental.pallas import tpu as pltpu
from jax.experimental.pallas import tpu_sc as plsc

B, S, D = 4, 8192, 128
FLAT = B * S
NC, NS = 2, 16
NW = NC * NS
PER_W = FLAT // NW
CHUNK = 128
N_CHUNKS = PER_W // CHUNK


NBUF = 3


def _emb_body(ids_hbm, tok_hbm, pos_hbm, out_hbm, idx_v, row_v,
              lsem, gsem0, gsem1, gsem2, ssem0, ssem1, ssem2):
    gsems = [gsem0, gsem1, gsem2]
    ssems = [ssem0, ssem1, ssem2]
    wid = lax.axis_index("s") * NC + lax.axis_index("c")
    base_w = wid * PER_W

    def start_loads(c):
        slot = c % NBUF
        base = base_w + c * CHUNK
        s0 = lax.rem(base, S)
        a = pltpu.async_copy(ids_hbm.at[pl.ds(base, CHUNK)], idx_v.at[slot], lsem)
        b = pltpu.async_copy(pos_hbm.at[pl.ds(s0, CHUNK)], row_v.at[slot], lsem)
        return a, b

    loads = {0: start_loads(0)}
    gathers = {}
    stores = {}
    for c in range(N_CHUNKS):
        slot = c % NBUF
        for d in loads.pop(c):
            d.wait()
        gathers[c] = pltpu.async_copy(tok_hbm.at[idx_v.at[slot]],
                                      row_v.at[slot], gsems[slot], add=True)
        if c + 1 < N_CHUNKS:
            if c - 2 >= 0:
                stores.pop(c - 2).wait()
            loads[c + 1] = start_loads(c + 1)
        if c - 1 >= 0:
            gathers.pop(c - 1).wait()
            pslot = (c - 1) % NBUF
            pbase = base_w + (c - 1) * CHUNK
            stores[c - 1] = pltpu.async_copy(
                row_v.at[pslot], out_hbm.at[pl.ds(pbase, CHUNK)], ssems[pslot])
    last = N_CHUNKS - 1
    gathers.pop(last).wait()
    lslot = last % NBUF
    lbase = base_w + last * CHUNK
    stores[last] = pltpu.async_copy(
        row_v.at[lslot], out_hbm.at[pl.ds(lbase, CHUNK)], ssems[lslot])
    for c in sorted(stores):
        stores.pop(c).wait()


def kernel(token_ids, tok_emb, pos_emb):
    ids = token_ids.reshape(FLAT).astype(jnp.int32)
    mesh = plsc.VectorSubcoreMesh(
        core_axis_name="c", subcore_axis_name="s",
        num_cores=NC, num_subcores=NS,
    )
    out = pl.kernel(
        _emb_body,
        out_type=jax.ShapeDtypeStruct((FLAT, D), jnp.float32),
        mesh=mesh,
        scratch_types=[
            pltpu.VMEM((NBUF, CHUNK), jnp.int32),
            pltpu.VMEM((NBUF, CHUNK, D), jnp.float32),
            pltpu.SemaphoreType.DMA,
            pltpu.SemaphoreType.DMA,
            pltpu.SemaphoreType.DMA,
            pltpu.SemaphoreType.DMA,
            pltpu.SemaphoreType.DMA,
            pltpu.SemaphoreType.DMA,
            pltpu.SemaphoreType.DMA,
        ],
    )(ids, tok_emb, pos_emb)
    return out.reshape(B, S, D)


# speedup vs baseline: 1.3915x; 1.0500x over previous
import jax
import jax.numpy as jnp
from jax import lax
from jax.experimental import pallas as pl
from jax.experimental.pallas import tpu as pltpu
from jax.experimental.pallas import tpu_sc as plsc

B, S, D = 4, 8192, 128
FLAT = B * S
NC, NS = 2, 16
NW = NC * NS
PER_W = FLAT // NW
CHUNK = 128
N_CHUNKS = PER_W // CHUNK


NBUF = 5
GDEPTH = 3


def _emb_body(ids_hbm, tok_hbm, pos_hbm, out_hbm, idx_v, row_v,
              isem, lsem, gsem0, gsem1, gsem2, gsem3, gsem4,
              ssem0, ssem1, ssem2, ssem3, ssem4):
    gsems = [gsem0, gsem1, gsem2, gsem3, gsem4]
    ssems = [ssem0, ssem1, ssem2, ssem3, ssem4]
    wid = lax.axis_index("s") * NC + lax.axis_index("c")
    base_w = wid * PER_W
    crow0 = wid * N_CHUNKS

    idx_all = pltpu.async_copy(ids_hbm.at[pl.ds(crow0, N_CHUNKS)], idx_v, isem)

    def start_load(c):
        slot = c % NBUF
        s0 = lax.rem(base_w + c * CHUNK, S)
        return pltpu.async_copy(pos_hbm.at[pl.ds(s0, CHUNK)], row_v.at[slot], lsem)

    def start_store(c):
        slot = c % NBUF
        base = base_w + c * CHUNK
        return pltpu.async_copy(row_v.at[slot], out_hbm.at[pl.ds(base, CHUNK)],
                                ssems[slot])

    loads = {0: start_load(0), 1: start_load(1)}
    gathers = {}
    stores = {}
    idx_all.wait()
    for c in range(N_CHUNKS):
        slot = c % NBUF
        loads.pop(c).wait()
        gathers[c] = pltpu.async_copy(tok_hbm.at[idx_v.at[c]],
                                      row_v.at[slot], gsems[slot], add=True)
        if c + 2 < N_CHUNKS:
            if c - GDEPTH >= 0:
                stores.pop(c - GDEPTH).wait()
            loads[c + 2] = start_load(c + 2)
        if c - (GDEPTH - 1) >= 0:
            p = c - (GDEPTH - 1)
            gathers.pop(p).wait()
            stores[p] = start_store(p)
    for p in range(N_CHUNKS - (GDEPTH - 1), N_CHUNKS):
        gathers.pop(p).wait()
        stores[p] = start_store(p)
    for c in sorted(stores):
        stores.pop(c).wait()


def kernel(token_ids, tok_emb, pos_emb):
    ids = token_ids.reshape(FLAT // CHUNK, CHUNK).astype(jnp.int32)
    mesh = plsc.VectorSubcoreMesh(
        core_axis_name="c", subcore_axis_name="s",
        num_cores=NC, num_subcores=NS,
    )
    out = pl.kernel(
        _emb_body,
        out_type=jax.ShapeDtypeStruct((FLAT, D), jnp.float32),
        mesh=mesh,
        scratch_types=(
            [pltpu.VMEM((N_CHUNKS, CHUNK), jnp.int32),
             pltpu.VMEM((NBUF, CHUNK, D), jnp.float32)]
            + [pltpu.SemaphoreType.DMA] * (2 + 2 * NBUF)
        ),
    )(ids, tok_emb, pos_emb)
    return out.reshape(B, S, D)


# speedup vs baseline: 1.3927x; 1.0009x over previous
import jax
import jax.numpy as jnp
from jax import lax
from jax.experimental import pallas as pl
from jax.experimental.pallas import tpu as pltpu
from jax.experimental.pallas import tpu_sc as plsc

B, S, D = 4, 8192, 128
FLAT = B * S
NC, NS = 2, 16
NW = NC * NS
PER_W = FLAT // NW
CHUNK = 128
N_CHUNKS = PER_W // CHUNK


NBUF = 5
GDEPTH = 3


def _emb_body(ids_hbm, tok_hbm, pos_hbm, out_hbm, idx_v, row_v,
              isem, lsem, gsem0, gsem1, gsem2, gsem3, gsem4,
              ssem0, ssem1, ssem2, ssem3, ssem4):
    gsems = [gsem0, gsem1, gsem2, gsem3, gsem4]
    ssems = [ssem0, ssem1, ssem2, ssem3, ssem4]
    wid = lax.axis_index("s") * NC + lax.axis_index("c")
    base_w = wid * PER_W
    wb = wid // (S // PER_W)
    woff = lax.rem(base_w, S)

    idx_all = pltpu.async_copy(ids_hbm.at[wb, pl.ds(woff, PER_W)], idx_v, isem)

    def start_load(c):
        slot = c % NBUF
        s0 = lax.rem(base_w + c * CHUNK, S)
        return pltpu.async_copy(pos_hbm.at[pl.ds(s0, CHUNK)], row_v.at[slot], lsem)

    def start_store(c):
        slot = c % NBUF
        base = base_w + c * CHUNK
        return pltpu.async_copy(row_v.at[slot], out_hbm.at[pl.ds(base, CHUNK)],
                                ssems[slot])

    loads = {0: start_load(0), 1: start_load(1)}
    gathers = {}
    stores = {}
    idx_all.wait()
    for c in range(N_CHUNKS):
        slot = c % NBUF
        loads.pop(c).wait()
        gathers[c] = pltpu.async_copy(tok_hbm.at[idx_v.at[pl.ds(c * CHUNK, CHUNK)]],
                                      row_v.at[slot], gsems[slot], add=True)
        if c + 2 < N_CHUNKS:
            if c - GDEPTH >= 0:
                stores.pop(c - GDEPTH).wait()
            loads[c + 2] = start_load(c + 2)
        if c - (GDEPTH - 1) >= 0:
            p = c - (GDEPTH - 1)
            gathers.pop(p).wait()
            stores[p] = start_store(p)
    for p in range(N_CHUNKS - (GDEPTH - 1), N_CHUNKS):
        gathers.pop(p).wait()
        stores[p] = start_store(p)
    for c in sorted(stores):
        stores.pop(c).wait()


def kernel(token_ids, tok_emb, pos_emb):
    ids = token_ids.astype(jnp.int32)
    mesh = plsc.VectorSubcoreMesh(
        core_axis_name="c", subcore_axis_name="s",
        num_cores=NC, num_subcores=NS,
    )
    out = pl.kernel(
        _emb_body,
        out_type=jax.ShapeDtypeStruct((FLAT, D), jnp.float32),
        mesh=mesh,
        scratch_types=(
            [pltpu.VMEM((PER_W,), jnp.int32),
             pltpu.VMEM((NBUF, CHUNK, D), jnp.float32)]
            + [pltpu.SemaphoreType.DMA] * (2 + 2 * NBUF)
        ),
    )(ids, tok_emb, pos_emb)
    return out.reshape(B, S, D)


# speedup vs baseline: 1.5194x; 1.0910x over previous
import jax
import jax.numpy as jnp
from jax import lax
from jax.experimental import pallas as pl
from jax.experimental.pallas import tpu as pltpu
from jax.experimental.pallas import tpu_sc as plsc

B, S, D = 4, 8192, 128
FLAT = B * S
NC, NS = 2, 16
NW = NC * NS
PER_W = FLAT // NW
CHUNK = 128
N_CHUNKS = PER_W // CHUNK


NBUF = 5
GDEPTH = 3
SPAN = FLAT // NW // B
JCH = SPAN // CHUNK


def _emb_body(ids_hbm, tok_hbm, pos_hbm, out_hbm, idx_v, pos_sh, row_v,
              isem, psem, lsem, gsem0, gsem1, gsem2, gsem3, gsem4,
              ssem0, ssem1, ssem2, ssem3, ssem4):
    gsems = [gsem0, gsem1, gsem2, gsem3, gsem4]
    ssems = [ssem0, ssem1, ssem2, ssem3, ssem4]
    sid = lax.axis_index("s")
    wid = sid * NC + lax.axis_index("c")
    s_base = wid * SPAN

    pos_stage = pltpu.async_copy(pos_hbm.at[pl.ds(s_base, SPAN)],
                                 pos_sh.at[sid], psem)
    idx_loads = [
        pltpu.async_copy(ids_hbm.at[b, pl.ds(s_base, SPAN)],
                         idx_v.at[pl.ds(b * SPAN, SPAN)], isem)
        for b in range(B)
    ]

    def chunk_coords(c):
        b, j = divmod(c, JCH)
        flat_base = b * S + s_base + j * CHUNK
        return b, j, flat_base

    def start_init(c):
        slot = c % NBUF
        _, j, _ = chunk_coords(c)
        return pltpu.async_copy(pos_sh.at[sid, pl.ds(j * CHUNK, CHUNK)],
                                row_v.at[slot], lsem)

    def start_store(c):
        slot = c % NBUF
        _, _, flat_base = chunk_coords(c)
        return pltpu.async_copy(row_v.at[slot],
                                out_hbm.at[pl.ds(flat_base, CHUNK)], ssems[slot])

    pos_stage.wait()
    for d in idx_loads:
        d.wait()
    inits = {0: start_init(0), 1: start_init(1)}
    gathers = {}
    stores = {}
    for c in range(N_CHUNKS):
        slot = c % NBUF
        b, j, _ = chunk_coords(c)
        inits.pop(c).wait()
        gathers[c] = pltpu.async_copy(
            tok_hbm.at[idx_v.at[pl.ds((b * JCH + j) * CHUNK, CHUNK)]],
            row_v.at[slot], gsems[slot], add=True)
        if c + 2 < N_CHUNKS:
            if c - GDEPTH >= 0:
                stores.pop(c - GDEPTH).wait()
            inits[c + 2] = start_init(c + 2)
        if c - (GDEPTH - 1) >= 0:
            p = c - (GDEPTH - 1)
            gathers.pop(p).wait()
            stores[p] = start_store(p)
    for p in range(N_CHUNKS - (GDEPTH - 1), N_CHUNKS):
        gathers.pop(p).wait()
        stores[p] = start_store(p)
    for c in sorted(stores):
        stores.pop(c).wait()


def kernel(token_ids, tok_emb, pos_emb):
    ids = token_ids.astype(jnp.int32)
    mesh = plsc.VectorSubcoreMesh(
        core_axis_name="c", subcore_axis_name="s",
        num_cores=NC, num_subcores=NS,
    )
    out = pl.kernel(
        _emb_body,
        out_type=jax.ShapeDtypeStruct((FLAT, D), jnp.float32),
        mesh=mesh,
        scratch_types=(
            [pltpu.VMEM((PER_W,), jnp.int32),
             pltpu.VMEM_SHARED((NS, SPAN, D), jnp.float32),
             pltpu.VMEM((NBUF, CHUNK, D), jnp.float32)]
            + [pltpu.SemaphoreType.DMA] * (3 + 2 * NBUF)
        ),
    )(ids, tok_emb, pos_emb)
    return out.reshape(B, S, D)
